# baseline (speedup 1.0000x reference)
import functools

import jax
import jax.numpy as jnp
from jax import lax
from jax.experimental import pallas as pl
from jax.experimental.pallas import tpu as pltpu
from jax.experimental.pallas import tpu_sc as plsc

N = 16384
M = 4096
K = 16
C_IN = 32
C_OUT = 64
NW = 32
QPW = M // NW
NCHUNK = N // 16
BIG = 3.0e38


def _fps_body(px_ref, py_ref, pz_ref, idx_ref, mind_ref):
    i32 = jnp.int32
    colio1 = lax.broadcasted_iota(i32, (1, 128), 1)
    rowio = lax.broadcasted_iota(i32, (128, 128), 0)
    colio = lax.broadcasted_iota(i32, (128, 128), 1)
    flat = rowio * 128 + colio

    idx_ref[0] = 0
    cm0 = colio1 == 0
    qx0 = jnp.sum(jnp.where(cm0, px_ref[pl.ds(0, 1), :], 0.0))
    qy0 = jnp.sum(jnp.where(cm0, py_ref[pl.ds(0, 1), :], 0.0))
    qz0 = jnp.sum(jnp.where(cm0, pz_ref[pl.ds(0, 1), :], 0.0))
    mind_ref[...] = jnp.full((128, 128), BIG, jnp.float32)

    def step(i, carry):
        qx, qy, qz = carry
        dx = px_ref[...] - qx
        dy = py_ref[...] - qy
        dz = pz_ref[...] - qz
        nmd = jnp.minimum(mind_ref[...], (dx * dx + dy * dy) + dz * dz)
        mind_ref[...] = nmd
        m = jnp.max(nmd)
        nxt = jnp.min(jnp.where(nmd == m, flat, i32(2**30)))
        idx_ref[i] = nxt
        sel = flat == nxt
        nqx = jnp.min(jnp.where(sel, px_ref[...], BIG))
        nqy = jnp.min(jnp.where(sel, py_ref[...], BIG))
        nqz = jnp.min(jnp.where(sel, pz_ref[...], BIG))
        return (nqx, nqy, nqz)

    lax.fori_loop(1, M, step, (qx0, qy0, qz0))


def _fps(px2, py2, pz2):
    return pl.pallas_call(
        _fps_body,
        out_shape=jax.ShapeDtypeStruct((M,), jnp.int32),
        in_specs=[pl.BlockSpec(memory_space=pltpu.VMEM)] * 3,
        out_specs=pl.BlockSpec(memory_space=pltpu.SMEM),
        scratch_shapes=[pltpu.VMEM((128, 128), jnp.float32)],
    )(px2, py2, pz2)


def _prep_body(x_ref, p1_ref, p1t_ref, wx_ref, wp_ref, b_ref, a_ref, aux_ref):
    xw = jnp.dot(x_ref[...].astype(jnp.bfloat16),
                 wx_ref[...].astype(jnp.bfloat16),
                 preferred_element_type=jnp.float32)
    pxc = p1_ref[:, 0:1]
    pyc = p1_ref[:, 1:2]
    pzc = p1_ref[:, 2:3]
    pw = (pxc * wp_ref[0:1, :] + pyc * wp_ref[1:2, :]) + pzc * wp_ref[2:3, :]
    a_ref[...] = (xw + pw) + b_ref[...]
    pxr = p1t_ref[0:1, :]
    pyr = p1t_ref[1:2, :]
    pzr = p1t_ref[2:3, :]
    aux_ref[0:1, :] = pxr.astype(jnp.bfloat16).astype(jnp.float32)
    aux_ref[1:2, :] = pyr.astype(jnp.bfloat16).astype(jnp.float32)
    aux_ref[2:3, :] = pzr.astype(jnp.bfloat16).astype(jnp.float32)
    aux_ref[3:4, :] = (pxr * pxr + pyr * pyr) + pzr * pzr


def _prep(x, p1, p1t, wx, wp, b_row):
    blk = 2048
    return pl.pallas_call(
        _prep_body,
        grid=(N // blk,),
        out_shape=[
            jax.ShapeDtypeStruct((N, C_OUT), jnp.float32),
            jax.ShapeDtypeStruct((4, N), jnp.float32),
        ],
        in_specs=[
            pl.BlockSpec((blk, C_IN), lambda i: (i, 0)),
            pl.BlockSpec((blk, 3), lambda i: (i, 0)),
            pl.BlockSpec((3, blk), lambda i: (0, i)),
            pl.BlockSpec((C_IN, C_OUT), lambda i: (0, 0)),
            pl.BlockSpec((3, C_OUT), lambda i: (0, 0)),
            pl.BlockSpec((1, C_OUT), lambda i: (0, 0)),
        ],
        out_specs=[
            pl.BlockSpec((blk, C_OUT), lambda i: (i, 0)),
            pl.BlockSpec((4, blk), lambda i: (0, i)),
        ],
    )(x, p1, p1t, wx, wp, b_row)


def _sc_knn(px, py, pz, aux, fidx, a_tab, wpf):
    mesh = plsc.VectorSubcoreMesh(core_axis_name="c", subcore_axis_name="s")

    @functools.partial(
        pl.kernel,
        mesh=mesh,
        compiler_params=pltpu.CompilerParams(
            needs_layout_passes=False, use_tc_tiling_on_sc=False),
        out_type=[
            jax.ShapeDtypeStruct((M * C_OUT,), jnp.float32),
            jax.ShapeDtypeStruct((3 * M,), jnp.float32),
        ],
        scratch_types=[
            pltpu.VMEM((N,), jnp.float32),
            pltpu.VMEM((N,), jnp.float32),
            pltpu.VMEM((N,), jnp.float32),
            pltpu.VMEM((N,), jnp.float32),
            pltpu.VMEM((QPW,), jnp.int32),
            pltpu.VMEM((192,), jnp.float32),
            pltpu.VMEM((K, C_OUT), jnp.float32),
            pltpu.VMEM((QPW * C_OUT,), jnp.float32),
            pltpu.VMEM((QPW,), jnp.float32),
            pltpu.VMEM((QPW,), jnp.float32),
            pltpu.VMEM((QPW,), jnp.float32),
            pltpu.VMEM((16,), jnp.float32),
            pltpu.VMEM((16,), jnp.float32),
            pltpu.VMEM((16,), jnp.int32),
            pltpu.SMEM((1,), jnp.float32),
            pltpu.SemaphoreType.DMA,
        ],
    )
    def body(px_h, py_h, pz_h, aux_h, fidx_h, a_h, wpf_h, y_h, fc_h,
             pxb_v, pyb_v, pzb_v, p2_v, fidx_v, wpf_v, arows_v, orow_v,
             qx_v, qy_v, qz_v, qg_v, td_v, ti_v, thr_s, sem):
        i32 = jnp.int32
        cid = lax.axis_index("c")
        sid = lax.axis_index("s")
        wid = sid * 2 + cid
        base = wid * QPW

        pltpu.sync_copy(aux_h.at[pl.ds(0 * N, N)], pxb_v)
        pltpu.sync_copy(aux_h.at[pl.ds(1 * N, N)], pyb_v)
        pltpu.sync_copy(aux_h.at[pl.ds(2 * N, N)], pzb_v)
        pltpu.sync_copy(aux_h.at[pl.ds(3 * N, N)], p2_v)
        pltpu.sync_copy(fidx_h.at[pl.ds(base, QPW)], fidx_v)
        pltpu.sync_copy(wpf_h, wpf_v)

        lanes = lax.iota(i32, 16)

        def group(g, carry):
            goff = g * 16
            qidx = fidx_v[pl.ds(goff, 16)]
            pltpu.async_copy(px_h.at[qidx], qg_v, sem).wait()
            qx16 = qg_v[...]
            pltpu.async_copy(py_h.at[qidx], qg_v, sem).wait()
            qy16 = qg_v[...]
            pltpu.async_copy(pz_h.at[qidx], qg_v, sem).wait()
            qz16 = qg_v[...]
            qx_v[pl.ds(goff, 16)] = qx16
            qy_v[pl.ds(goff, 16)] = qy16
            qz_v[pl.ds(goff, 16)] = qz16
            pltpu.async_copy(aux_h.at[qidx], qg_v, sem).wait()
            qxb16 = qg_v[...]
            pltpu.async_copy(aux_h.at[qidx + N], qg_v, sem).wait()
            qyb16 = qg_v[...]
            pltpu.async_copy(aux_h.at[qidx + 2 * N], qg_v, sem).wait()
            qzb16 = qg_v[...]

            def query(l, carry2):
                lm = lanes == l
                qx = jnp.max(jnp.where(lm, qx16, -BIG))
                qy = jnp.max(jnp.where(lm, qy16, -BIG))
                qz = jnp.max(jnp.where(lm, qz16, -BIG))
                qxb = jnp.max(jnp.where(lm, qxb16, -BIG))
                qyb = jnp.max(jnp.where(lm, qyb16, -BIG))
                qzb = jnp.max(jnp.where(lm, qzb16, -BIG))
                q2 = (qx * qx + qy * qy) + qz * qz
                m2x = -2.0 * qxb
                m2y = -2.0 * qyb
                m2z = -2.0 * qzb

                td_v[...] = jnp.full((16,), BIG, jnp.float32)
                ti_v[...] = jnp.zeros((16,), i32)
                thr_s[0] = BIG

                def dist_at(s):
                    pxc = pxb_v[pl.ds(s, 16)]
                    pyc = pyb_v[pl.ds(s, 16)]
                    pzc = pzb_v[pl.ds(s, 16)]
                    p2c = p2_v[pl.ds(s, 16)]
                    dotm = (m2x * pxc + m2y * pyc) + m2z * pzc
                    return (q2 + dotm) + p2c

                def merge(d2, s, t_d, t_i):
                    ic = lanes + s
                    ca, cia = plsc.sort_key_val(d2, ic, descending=False)
                    take = ca < t_d
                    t2 = jnp.where(take, ca, t_d)
                    ti2 = jnp.where(take, cia, t_i)
                    return plsc.sort_key_val(t2, ti2, descending=True)

                NSUB = 8

                def chunk(j, carry3):
                    s = j * (16 * NSUB)
                    d2s = [dist_at(s + 16 * k) for k in range(NSUB)]
                    cmin = d2s[0]
                    for k in range(1, NSUB):
                        cmin = jnp.minimum(cmin, d2s[k])
                    thr = thr_s[0]

                    def do_merges(_, c):
                        t_d = td_v[...]
                        t_i = ti_v[...]
                        for k in range(NSUB):
                            t_d, t_i = merge(d2s[k], s + 16 * k, t_d, t_i)
                        td_v[...] = t_d
                        ti_v[...] = t_i
                        thr_s[0] = jnp.max(t_d)
                        return c

                    cnt = jnp.where(jnp.any(cmin < thr), 1, 0)
                    lax.fori_loop(0, cnt, do_merges, 0)
                    return carry3

                lax.fori_loop(0, NCHUNK // NSUB, chunk, 0)
                t_i = ti_v[...]

                pltpu.async_copy(a_h.at[t_i], arows_v, sem).wait()

                obase = g * (16 * C_OUT) + l * C_OUT
                for cc in range(C_OUT // 16):
                    mx = arows_v[0, pl.ds(cc * 16, 16)]
                    for r in range(1, K):
                        mx = jnp.maximum(mx, arows_v[r, pl.ds(cc * 16, 16)])
                    wp0 = wpf_v[pl.ds(0 * C_OUT + cc * 16, 16)]
                    wp1 = wpf_v[pl.ds(1 * C_OUT + cc * 16, 16)]
                    wp2 = wpf_v[pl.ds(2 * C_OUT + cc * 16, 16)]
                    tq = (qx * wp0 + qy * wp1) + qz * wp2
                    yv = jnp.maximum(mx - tq, 0.0)
                    orow_v[pl.ds(obase + cc * 16, 16)] = yv
                return carry2

            return lax.fori_loop(0, 16, query, carry)

        lax.fori_loop(0, QPW // 16, group, 0)

        pltpu.sync_copy(orow_v, y_h.at[pl.ds(base * C_OUT, QPW * C_OUT)])
        pltpu.sync_copy(qx_v, fc_h.at[pl.ds(0 * M + base, QPW)])
        pltpu.sync_copy(qy_v, fc_h.at[pl.ds(1 * M + base, QPW)])
        pltpu.sync_copy(qz_v, fc_h.at[pl.ds(2 * M + base, QPW)])

    return body(px, py, pz, aux, fidx, a_tab, wpf)


def kernel(x, p1, W_td, b_td, Wq, Wk, Wv, Wd1, bd1, Wd2, bd2,
           Wg1, bg1, Wg2, bg2, Wo, bo):
    px = p1[:, 0]
    py = p1[:, 1]
    pz = p1[:, 2]
    fidx = _fps(px.reshape(128, 128), py.reshape(128, 128),
                pz.reshape(128, 128))
    wx = W_td[:C_IN]
    wp = W_td[C_IN:C_IN + 3]
    a_tab, aux = _prep(x, p1, p1.T, wx, wp, b_td.reshape(1, C_OUT))
    yflat, fcflat = _sc_knn(px, py, pz, aux.reshape(4 * N), fidx, a_tab,
                            wp.reshape(192))
    y = yflat.reshape(M, C_OUT)
    fps_coords = fcflat.reshape(3, M).T
    return (y, fps_coords)


# speedup vs baseline: 1.1479x; 1.1479x over previous
import functools

import jax
import jax.numpy as jnp
from jax import lax
from jax.experimental import pallas as pl
from jax.experimental.pallas import tpu as pltpu
from jax.experimental.pallas import tpu_sc as plsc

N = 16384
M = 4096
K = 16
C_IN = 32
C_OUT = 64
NW = 32
QPW = M // NW
NCHUNK = N // 16
BIG = 3.0e38


def _fps_body(px_ref, py_ref, pz_ref, idx_ref, mind_ref):
    i32 = jnp.int32
    colio1 = lax.broadcasted_iota(i32, (1, 128), 1)
    rowio = lax.broadcasted_iota(i32, (128, 128), 0)
    colio = lax.broadcasted_iota(i32, (128, 128), 1)
    flat = rowio * 128 + colio

    def rmin2(a):
        return jnp.min(jnp.min(a, axis=0, keepdims=True), axis=1,
                       keepdims=True)

    idx_ref[0] = 0
    cm0 = colio1 == 0
    row0x = jnp.where(cm0, px_ref[pl.ds(0, 1), :], BIG)
    row0y = jnp.where(cm0, py_ref[pl.ds(0, 1), :], BIG)
    row0z = jnp.where(cm0, pz_ref[pl.ds(0, 1), :], BIG)
    qx0 = jnp.min(row0x, axis=1, keepdims=True).reshape(1, 1)
    qy0 = jnp.min(row0y, axis=1, keepdims=True).reshape(1, 1)
    qz0 = jnp.min(row0z, axis=1, keepdims=True).reshape(1, 1)
    mind_ref[...] = jnp.full((128, 128), BIG, jnp.float32)

    def step(i, carry):
        qx, qy, qz = carry
        dx = px_ref[...] - qx
        dy = py_ref[...] - qy
        dz = pz_ref[...] - qz
        nmd = jnp.minimum(mind_ref[...], (dx * dx + dy * dy) + dz * dz)
        mind_ref[...] = nmd
        m = jnp.max(jnp.max(nmd, axis=0, keepdims=True), axis=1,
                    keepdims=True)
        nxt = rmin2(jnp.where(nmd == m, flat, i32(2**30)))
        idx_ref[i] = nxt[0, 0]
        sel = flat == nxt
        nqx = rmin2(jnp.where(sel, px_ref[...], BIG))
        nqy = rmin2(jnp.where(sel, py_ref[...], BIG))
        nqz = rmin2(jnp.where(sel, pz_ref[...], BIG))
        return (nqx, nqy, nqz)

    lax.fori_loop(1, M, step, (qx0, qy0, qz0))


def _fps(px2, py2, pz2):
    return pl.pallas_call(
        _fps_body,
        out_shape=jax.ShapeDtypeStruct((M,), jnp.int32),
        in_specs=[pl.BlockSpec(memory_space=pltpu.VMEM)] * 3,
        out_specs=pl.BlockSpec(memory_space=pltpu.SMEM),
        scratch_shapes=[pltpu.VMEM((128, 128), jnp.float32)],
    )(px2, py2, pz2)


def _prep_body(x_ref, p1_ref, p1t_ref, wx_ref, wp_ref, b_ref, a_ref, aux_ref):
    xw = jnp.dot(x_ref[...].astype(jnp.bfloat16),
                 wx_ref[...].astype(jnp.bfloat16),
                 preferred_element_type=jnp.float32)
    pxc = p1_ref[:, 0:1]
    pyc = p1_ref[:, 1:2]
    pzc = p1_ref[:, 2:3]
    pw = (pxc * wp_ref[0:1, :] + pyc * wp_ref[1:2, :]) + pzc * wp_ref[2:3, :]
    a_ref[...] = (xw + pw) + b_ref[...]
    pxr = p1t_ref[0:1, :]
    pyr = p1t_ref[1:2, :]
    pzr = p1t_ref[2:3, :]
    aux_ref[0:1, :] = pxr.astype(jnp.bfloat16).astype(jnp.float32)
    aux_ref[1:2, :] = pyr.astype(jnp.bfloat16).astype(jnp.float32)
    aux_ref[2:3, :] = pzr.astype(jnp.bfloat16).astype(jnp.float32)
    aux_ref[3:4, :] = (pxr * pxr + pyr * pyr) + pzr * pzr


def _prep(x, p1, p1t, wx, wp, b_row):
    blk = 2048
    return pl.pallas_call(
        _prep_body,
        grid=(N // blk,),
        out_shape=[
            jax.ShapeDtypeStruct((N, C_OUT), jnp.float32),
            jax.ShapeDtypeStruct((4, N), jnp.float32),
        ],
        in_specs=[
            pl.BlockSpec((blk, C_IN), lambda i: (i, 0)),
            pl.BlockSpec((blk, 3), lambda i: (i, 0)),
            pl.BlockSpec((3, blk), lambda i: (0, i)),
            pl.BlockSpec((C_IN, C_OUT), lambda i: (0, 0)),
            pl.BlockSpec((3, C_OUT), lambda i: (0, 0)),
            pl.BlockSpec((1, C_OUT), lambda i: (0, 0)),
        ],
        out_specs=[
            pl.BlockSpec((blk, C_OUT), lambda i: (i, 0)),
            pl.BlockSpec((4, blk), lambda i: (0, i)),
        ],
    )(x, p1, p1t, wx, wp, b_row)


def _sc_knn(px, py, pz, aux, fidx, a_tab, wpf):
    mesh = plsc.VectorSubcoreMesh(core_axis_name="c", subcore_axis_name="s")

    @functools.partial(
        pl.kernel,
        mesh=mesh,
        compiler_params=pltpu.CompilerParams(
            needs_layout_passes=False, use_tc_tiling_on_sc=False),
        out_type=[
            jax.ShapeDtypeStruct((M * C_OUT,), jnp.float32),
            jax.ShapeDtypeStruct((3 * M,), jnp.float32),
        ],
        scratch_types=[
            pltpu.VMEM((N,), jnp.float32),
            pltpu.VMEM((N,), jnp.float32),
            pltpu.VMEM((N,), jnp.float32),
            pltpu.VMEM((N,), jnp.float32),
            pltpu.VMEM((QPW,), jnp.int32),
            pltpu.VMEM((192,), jnp.float32),
            pltpu.VMEM((K, C_OUT), jnp.float32),
            pltpu.VMEM((QPW * C_OUT,), jnp.float32),
            pltpu.VMEM((QPW,), jnp.float32),
            pltpu.VMEM((QPW,), jnp.float32),
            pltpu.VMEM((QPW,), jnp.float32),
            pltpu.VMEM((16,), jnp.float32),
            pltpu.VMEM((16,), jnp.float32),
            pltpu.VMEM((16,), jnp.int32),
            pltpu.SMEM((1,), jnp.float32),
            pltpu.SemaphoreType.DMA,
        ],
    )
    def body(px_h, py_h, pz_h, aux_h, fidx_h, a_h, wpf_h, y_h, fc_h,
             pxb_v, pyb_v, pzb_v, p2_v, fidx_v, wpf_v, arows_v, orow_v,
             qx_v, qy_v, qz_v, qg_v, td_v, ti_v, thr_s, sem):
        i32 = jnp.int32
        cid = lax.axis_index("c")
        sid = lax.axis_index("s")
        wid = sid * 2 + cid
        base = wid * QPW

        pltpu.sync_copy(aux_h.at[pl.ds(0 * N, N)], pxb_v)
        pltpu.sync_copy(aux_h.at[pl.ds(1 * N, N)], pyb_v)
        pltpu.sync_copy(aux_h.at[pl.ds(2 * N, N)], pzb_v)
        pltpu.sync_copy(aux_h.at[pl.ds(3 * N, N)], p2_v)
        pltpu.sync_copy(fidx_h.at[pl.ds(base, QPW)], fidx_v)
        pltpu.sync_copy(wpf_h, wpf_v)

        lanes = lax.iota(i32, 16)

        def group(g, carry):
            goff = g * 16
            qidx = fidx_v[pl.ds(goff, 16)]
            pltpu.async_copy(px_h.at[qidx], qg_v, sem).wait()
            qx16 = qg_v[...]
            pltpu.async_copy(py_h.at[qidx], qg_v, sem).wait()
            qy16 = qg_v[...]
            pltpu.async_copy(pz_h.at[qidx], qg_v, sem).wait()
            qz16 = qg_v[...]
            qx_v[pl.ds(goff, 16)] = qx16
            qy_v[pl.ds(goff, 16)] = qy16
            qz_v[pl.ds(goff, 16)] = qz16
            pltpu.async_copy(aux_h.at[qidx], qg_v, sem).wait()
            qxb16 = qg_v[...]
            pltpu.async_copy(aux_h.at[qidx + N], qg_v, sem).wait()
            qyb16 = qg_v[...]
            pltpu.async_copy(aux_h.at[qidx + 2 * N], qg_v, sem).wait()
            qzb16 = qg_v[...]

            def query(l, carry2):
                lm = lanes == l
                qx = jnp.max(jnp.where(lm, qx16, -BIG))
                qy = jnp.max(jnp.where(lm, qy16, -BIG))
                qz = jnp.max(jnp.where(lm, qz16, -BIG))
                qxb = jnp.max(jnp.where(lm, qxb16, -BIG))
                qyb = jnp.max(jnp.where(lm, qyb16, -BIG))
                qzb = jnp.max(jnp.where(lm, qzb16, -BIG))
                q2 = (qx * qx + qy * qy) + qz * qz
                m2x = -2.0 * qxb
                m2y = -2.0 * qyb
                m2z = -2.0 * qzb

                td_v[...] = jnp.full((16,), BIG, jnp.float32)
                ti_v[...] = jnp.zeros((16,), i32)
                thr_s[0] = BIG

                def dist_at(s):
                    pxc = pxb_v[pl.ds(s, 16)]
                    pyc = pyb_v[pl.ds(s, 16)]
                    pzc = pzb_v[pl.ds(s, 16)]
                    p2c = p2_v[pl.ds(s, 16)]
                    dotm = (m2x * pxc + m2y * pyc) + m2z * pzc
                    return (q2 + dotm) + p2c

                def merge(d2, s, t_d, t_i):
                    ic = lanes + s
                    ca, cia = plsc.sort_key_val(d2, ic, descending=False)
                    take = ca < t_d
                    t2 = jnp.where(take, ca, t_d)
                    ti2 = jnp.where(take, cia, t_i)
                    return plsc.sort_key_val(t2, ti2, descending=True)

                NSUB = 8

                def chunk(j, carry3):
                    s = j * (16 * NSUB)
                    d2s = [dist_at(s + 16 * k) for k in range(NSUB)]
                    cmin = d2s[0]
                    for k in range(1, NSUB):
                        cmin = jnp.minimum(cmin, d2s[k])
                    thr = thr_s[0]

                    def do_merges(_, c):
                        t_d = td_v[...]
                        t_i = ti_v[...]
                        for k in range(NSUB):
                            t_d, t_i = merge(d2s[k], s + 16 * k, t_d, t_i)
                        td_v[...] = t_d
                        ti_v[...] = t_i
                        thr_s[0] = jnp.max(t_d)
                        return c

                    cnt = jnp.where(jnp.any(cmin < thr), 1, 0)
                    lax.fori_loop(0, cnt, do_merges, 0)
                    return carry3

                lax.fori_loop(0, NCHUNK // NSUB, chunk, 0)
                t_i = ti_v[...]

                pltpu.async_copy(a_h.at[t_i], arows_v, sem).wait()

                obase = g * (16 * C_OUT) + l * C_OUT
                for cc in range(C_OUT // 16):
                    mx = arows_v[0, pl.ds(cc * 16, 16)]
                    for r in range(1, K):
                        mx = jnp.maximum(mx, arows_v[r, pl.ds(cc * 16, 16)])
                    wp0 = wpf_v[pl.ds(0 * C_OUT + cc * 16, 16)]
                    wp1 = wpf_v[pl.ds(1 * C_OUT + cc * 16, 16)]
                    wp2 = wpf_v[pl.ds(2 * C_OUT + cc * 16, 16)]
                    tq = (qx * wp0 + qy * wp1) + qz * wp2
                    yv = jnp.maximum(mx - tq, 0.0)
                    orow_v[pl.ds(obase + cc * 16, 16)] = yv
                return carry2

            return lax.fori_loop(0, 16, query, carry)

        lax.fori_loop(0, QPW // 16, group, 0)

        pltpu.sync_copy(orow_v, y_h.at[pl.ds(base * C_OUT, QPW * C_OUT)])
        pltpu.sync_copy(qx_v, fc_h.at[pl.ds(0 * M + base, QPW)])
        pltpu.sync_copy(qy_v, fc_h.at[pl.ds(1 * M + base, QPW)])
        pltpu.sync_copy(qz_v, fc_h.at[pl.ds(2 * M + base, QPW)])

    return body(px, py, pz, aux, fidx, a_tab, wpf)


def kernel(x, p1, W_td, b_td, Wq, Wk, Wv, Wd1, bd1, Wd2, bd2,
           Wg1, bg1, Wg2, bg2, Wo, bo):
    px = p1[:, 0]
    py = p1[:, 1]
    pz = p1[:, 2]
    fidx = _fps(px.reshape(128, 128), py.reshape(128, 128),
                pz.reshape(128, 128))
    wx = W_td[:C_IN]
    wp = W_td[C_IN:C_IN + 3]
    a_tab, aux = _prep(x, p1, p1.T, wx, wp, b_td.reshape(1, C_OUT))
    yflat, fcflat = _sc_knn(px, py, pz, aux.reshape(4 * N), fidx, a_tab,
                            wp.reshape(192))
    y = yflat.reshape(M, C_OUT)
    fps_coords = fcflat.reshape(3, M).T
    return (y, fps_coords)


# speedup vs baseline: 1.1675x; 1.0171x over previous
import functools

import jax
import jax.numpy as jnp
from jax import lax
from jax.experimental import pallas as pl
from jax.experimental.pallas import tpu as pltpu
from jax.experimental.pallas import tpu_sc as plsc

N = 16384
M = 4096
K = 16
C_IN = 32
C_OUT = 64
NW = 32
QPW = M // NW
NCHUNK = N // 16
BIG = 3.0e38


def _fps_body(px_ref, py_ref, pz_ref, idx_ref, mind_ref):
    i32 = jnp.int32
    colio1 = lax.broadcasted_iota(i32, (1, 128), 1)
    rowio = lax.broadcasted_iota(i32, (128, 128), 0)
    colio = lax.broadcasted_iota(i32, (128, 128), 1)
    flat = rowio * 128 + colio

    def rmin2(a):
        return jnp.min(jnp.min(a, axis=0, keepdims=True), axis=1,
                       keepdims=True)

    idx_ref[0] = 0
    cm0 = colio1 == 0
    row0x = jnp.where(cm0, px_ref[pl.ds(0, 1), :], BIG)
    row0y = jnp.where(cm0, py_ref[pl.ds(0, 1), :], BIG)
    row0z = jnp.where(cm0, pz_ref[pl.ds(0, 1), :], BIG)
    qx0 = jnp.min(row0x, axis=1, keepdims=True).reshape(1, 1)
    qy0 = jnp.min(row0y, axis=1, keepdims=True).reshape(1, 1)
    qz0 = jnp.min(row0z, axis=1, keepdims=True).reshape(1, 1)
    mind_ref[...] = jnp.full((128, 128), BIG, jnp.float32)

    def step(i, carry):
        qx, qy, qz = carry
        dx = px_ref[...] - qx
        dy = py_ref[...] - qy
        dz = pz_ref[...] - qz
        nmd = jnp.minimum(mind_ref[...], (dx * dx + dy * dy) + dz * dz)
        mind_ref[...] = nmd
        m = jnp.max(jnp.max(nmd, axis=0, keepdims=True), axis=1,
                    keepdims=True)
        nxt = rmin2(jnp.where(nmd == m, flat, i32(2**30)))
        idx_ref[i] = nxt[0, 0]
        sel = flat == nxt
        nqx = rmin2(jnp.where(sel, px_ref[...], BIG))
        nqy = rmin2(jnp.where(sel, py_ref[...], BIG))
        nqz = rmin2(jnp.where(sel, pz_ref[...], BIG))
        return (nqx, nqy, nqz)

    lax.fori_loop(1, M, step, (qx0, qy0, qz0))


def _fps(px2, py2, pz2):
    return pl.pallas_call(
        _fps_body,
        out_shape=jax.ShapeDtypeStruct((M,), jnp.int32),
        in_specs=[pl.BlockSpec(memory_space=pltpu.VMEM)] * 3,
        out_specs=pl.BlockSpec(memory_space=pltpu.SMEM),
        scratch_shapes=[pltpu.VMEM((128, 128), jnp.float32)],
    )(px2, py2, pz2)


def _prep_body(x_ref, p1_ref, p1t_ref, wx_ref, wp_ref, b_ref, a_ref, aux_ref):
    xw = jnp.dot(x_ref[...].astype(jnp.bfloat16),
                 wx_ref[...].astype(jnp.bfloat16),
                 preferred_element_type=jnp.float32)
    pxc = p1_ref[:, 0:1]
    pyc = p1_ref[:, 1:2]
    pzc = p1_ref[:, 2:3]
    pw = (pxc * wp_ref[0:1, :] + pyc * wp_ref[1:2, :]) + pzc * wp_ref[2:3, :]
    a_ref[...] = (xw + pw) + b_ref[...]
    pxr = p1t_ref[0:1, :]
    pyr = p1t_ref[1:2, :]
    pzr = p1t_ref[2:3, :]
    aux_ref[0:1, :] = pxr.astype(jnp.bfloat16).astype(jnp.float32)
    aux_ref[1:2, :] = pyr.astype(jnp.bfloat16).astype(jnp.float32)
    aux_ref[2:3, :] = pzr.astype(jnp.bfloat16).astype(jnp.float32)
    aux_ref[3:4, :] = (pxr * pxr + pyr * pyr) + pzr * pzr


def _prep(x, p1, p1t, wx, wp, b_row):
    blk = 2048
    return pl.pallas_call(
        _prep_body,
        grid=(N // blk,),
        out_shape=[
            jax.ShapeDtypeStruct((N, C_OUT), jnp.float32),
            jax.ShapeDtypeStruct((4, N), jnp.float32),
        ],
        in_specs=[
            pl.BlockSpec((blk, C_IN), lambda i: (i, 0)),
            pl.BlockSpec((blk, 3), lambda i: (i, 0)),
            pl.BlockSpec((3, blk), lambda i: (0, i)),
            pl.BlockSpec((C_IN, C_OUT), lambda i: (0, 0)),
            pl.BlockSpec((3, C_OUT), lambda i: (0, 0)),
            pl.BlockSpec((1, C_OUT), lambda i: (0, 0)),
        ],
        out_specs=[
            pl.BlockSpec((blk, C_OUT), lambda i: (i, 0)),
            pl.BlockSpec((4, blk), lambda i: (0, i)),
        ],
    )(x, p1, p1t, wx, wp, b_row)


def _sc_knn(px, py, pz, aux, fidx, a_tab, wpf):
    mesh = plsc.VectorSubcoreMesh(core_axis_name="c", subcore_axis_name="s")

    @functools.partial(
        pl.kernel,
        mesh=mesh,
        compiler_params=pltpu.CompilerParams(
            needs_layout_passes=False, use_tc_tiling_on_sc=False),
        out_type=[
            jax.ShapeDtypeStruct((M * C_OUT,), jnp.float32),
            jax.ShapeDtypeStruct((3 * M,), jnp.float32),
        ],
        scratch_types=[
            pltpu.VMEM((N,), jnp.float32),
            pltpu.VMEM((N,), jnp.float32),
            pltpu.VMEM((N,), jnp.float32),
            pltpu.VMEM((N,), jnp.float32),
            pltpu.VMEM((QPW,), jnp.int32),
            pltpu.VMEM((192,), jnp.float32),
            pltpu.VMEM((K, C_OUT), jnp.float32),
            pltpu.VMEM((QPW * C_OUT,), jnp.float32),
            pltpu.VMEM((QPW,), jnp.float32),
            pltpu.VMEM((QPW,), jnp.float32),
            pltpu.VMEM((QPW,), jnp.float32),
            pltpu.VMEM((16,), jnp.float32),
            pltpu.VMEM((16,), jnp.float32),
            pltpu.VMEM((16,), jnp.int32),
            pltpu.SMEM((1,), jnp.float32),
            pltpu.SemaphoreType.DMA,
        ],
    )
    def body(px_h, py_h, pz_h, aux_h, fidx_h, a_h, wpf_h, y_h, fc_h,
             pxb_v, pyb_v, pzb_v, p2_v, fidx_v, wpf_v, arows_v, orow_v,
             qx_v, qy_v, qz_v, qg_v, td_v, ti_v, thr_s, sem):
        i32 = jnp.int32
        cid = lax.axis_index("c")
        sid = lax.axis_index("s")
        wid = sid * 2 + cid
        base = wid * QPW

        pltpu.sync_copy(aux_h.at[pl.ds(0 * N, N)], pxb_v)
        pltpu.sync_copy(aux_h.at[pl.ds(1 * N, N)], pyb_v)
        pltpu.sync_copy(aux_h.at[pl.ds(2 * N, N)], pzb_v)
        pltpu.sync_copy(aux_h.at[pl.ds(3 * N, N)], p2_v)
        pltpu.sync_copy(fidx_h.at[pl.ds(base, QPW)], fidx_v)
        pltpu.sync_copy(wpf_h, wpf_v)

        lanes = lax.iota(i32, 16)

        def group(g, carry):
            goff = g * 16
            qidx = fidx_v[pl.ds(goff, 16)]
            pltpu.async_copy(px_h.at[qidx], qg_v, sem).wait()
            qx16 = qg_v[...]
            pltpu.async_copy(py_h.at[qidx], qg_v, sem).wait()
            qy16 = qg_v[...]
            pltpu.async_copy(pz_h.at[qidx], qg_v, sem).wait()
            qz16 = qg_v[...]
            qx_v[pl.ds(goff, 16)] = qx16
            qy_v[pl.ds(goff, 16)] = qy16
            qz_v[pl.ds(goff, 16)] = qz16
            pltpu.async_copy(aux_h.at[qidx], qg_v, sem).wait()
            qxb16 = qg_v[...]
            pltpu.async_copy(aux_h.at[qidx + N], qg_v, sem).wait()
            qyb16 = qg_v[...]
            pltpu.async_copy(aux_h.at[qidx + 2 * N], qg_v, sem).wait()
            qzb16 = qg_v[...]

            def query(l, carry2):
                lm = lanes == l
                qx = jnp.max(jnp.where(lm, qx16, -BIG))
                qy = jnp.max(jnp.where(lm, qy16, -BIG))
                qz = jnp.max(jnp.where(lm, qz16, -BIG))
                qxb = jnp.max(jnp.where(lm, qxb16, -BIG))
                qyb = jnp.max(jnp.where(lm, qyb16, -BIG))
                qzb = jnp.max(jnp.where(lm, qzb16, -BIG))
                q2 = (qx * qx + qy * qy) + qz * qz
                m2x = -2.0 * qxb
                m2y = -2.0 * qyb
                m2z = -2.0 * qzb

                td_v[...] = jnp.full((16,), BIG, jnp.float32)
                ti_v[...] = jnp.zeros((16,), i32)
                thr_s[0] = BIG

                def dist_at(s):
                    pxc = pxb_v[pl.ds(s, 16)]
                    pyc = pyb_v[pl.ds(s, 16)]
                    pzc = pzb_v[pl.ds(s, 16)]
                    p2c = p2_v[pl.ds(s, 16)]
                    dotm = (m2x * pxc + m2y * pyc) + m2z * pzc
                    return (q2 + dotm) + p2c

                def merge(d2, s, t_d, t_i):
                    ic = lanes + s
                    ca, cia = plsc.sort_key_val(d2, ic, descending=False)
                    take = ca < t_d
                    t2 = jnp.where(take, ca, t_d)
                    ti2 = jnp.where(take, cia, t_i)
                    return plsc.sort_key_val(t2, ti2, descending=True)

                NSUB = 16

                def chunk(j, carry3):
                    s = j * (16 * NSUB)
                    d2s = [dist_at(s + 16 * k) for k in range(NSUB)]
                    cmin = d2s[0]
                    for k in range(1, NSUB):
                        cmin = jnp.minimum(cmin, d2s[k])
                    thr = thr_s[0]

                    def do_merges(_, c):
                        t_d = td_v[...]
                        t_i = ti_v[...]
                        for k in range(NSUB):
                            t_d, t_i = merge(d2s[k], s + 16 * k, t_d, t_i)
                        td_v[...] = t_d
                        ti_v[...] = t_i
                        thr_s[0] = jnp.max(t_d)
                        return c

                    cnt = jnp.where(jnp.any(cmin < thr), 1, 0)
                    lax.fori_loop(0, cnt, do_merges, 0)
                    return carry3

                lax.fori_loop(0, NCHUNK // NSUB, chunk, 0)
                t_i = ti_v[...]

                pltpu.async_copy(a_h.at[t_i], arows_v, sem).wait()

                obase = g * (16 * C_OUT) + l * C_OUT
                for cc in range(C_OUT // 16):
                    mx = arows_v[0, pl.ds(cc * 16, 16)]
                    for r in range(1, K):
                        mx = jnp.maximum(mx, arows_v[r, pl.ds(cc * 16, 16)])
                    wp0 = wpf_v[pl.ds(0 * C_OUT + cc * 16, 16)]
                    wp1 = wpf_v[pl.ds(1 * C_OUT + cc * 16, 16)]
                    wp2 = wpf_v[pl.ds(2 * C_OUT + cc * 16, 16)]
                    tq = (qx * wp0 + qy * wp1) + qz * wp2
                    yv = jnp.maximum(mx - tq, 0.0)
                    orow_v[pl.ds(obase + cc * 16, 16)] = yv
                return carry2

            return lax.fori_loop(0, 16, query, carry)

        lax.fori_loop(0, QPW // 16, group, 0)

        pltpu.sync_copy(orow_v, y_h.at[pl.ds(base * C_OUT, QPW * C_OUT)])
        pltpu.sync_copy(qx_v, fc_h.at[pl.ds(0 * M + base, QPW)])
        pltpu.sync_copy(qy_v, fc_h.at[pl.ds(1 * M + base, QPW)])
        pltpu.sync_copy(qz_v, fc_h.at[pl.ds(2 * M + base, QPW)])

    return body(px, py, pz, aux, fidx, a_tab, wpf)


def kernel(x, p1, W_td, b_td, Wq, Wk, Wv, Wd1, bd1, Wd2, bd2,
           Wg1, bg1, Wg2, bg2, Wo, bo):
    px = p1[:, 0]
    py = p1[:, 1]
    pz = p1[:, 2]
    fidx = _fps(px.reshape(128, 128), py.reshape(128, 128),
                pz.reshape(128, 128))
    wx = W_td[:C_IN]
    wp = W_td[C_IN:C_IN + 3]
    a_tab, aux = _prep(x, p1, p1.T, wx, wp, b_td.reshape(1, C_OUT))
    yflat, fcflat = _sc_knn(px, py, pz, aux.reshape(4 * N), fidx, a_tab,
                            wp.reshape(192))
    y = yflat.reshape(M, C_OUT)
    fps_coords = fcflat.reshape(3, M).T
    return (y, fps_coords)
